# Initial kernel scaffold; baseline (speedup 1.0000x reference)
import functools

import jax
import jax.numpy as jnp
from jax import lax
from jax.experimental import pallas as pl
from jax.experimental.pallas import tpu as pltpu
from jax.experimental.pallas import tpu_sc as plsc

NUM_GRAPHS = 1024
HIDDEN = 14
N_NODES = 100000

NC = 2
NS = 16
NW = NC * NS
CHUNK = 3200
G = CHUNK // 128
N_PAD = NW * CHUNK
ROWS_PER_TILE = NUM_GRAPHS // NS


def _seg_body(x_hbm, idx_hbm, z14_hbm, z1_hbm, ones_hbm,
              out_s, out_c, xv, idxv, onesv, accs, accc):
    cid = lax.axis_index("c")
    sid = lax.axis_index("s")
    wid = cid * NS + sid
    base = wid * CHUNK

    pltpu.sync_copy(z14_hbm, accs.at[pl.ds(sid * ROWS_PER_TILE, ROWS_PER_TILE), :])
    pltpu.sync_copy(z1_hbm, accc.at[pl.ds(sid * ROWS_PER_TILE, ROWS_PER_TILE), :])
    pltpu.sync_copy(ones_hbm, onesv)

    pltpu.sync_copy(idx_hbm.at[pl.ds(wid * G, G), :], idxv)

    last_real = N_NODES - (NW - 1) * CHUNK

    @pl.when(wid < NW - 1)
    def _():
        pltpu.sync_copy(x_hbm.at[pl.ds(base, CHUNK), :], xv.at[pl.ds(0, CHUNK), :])

    @pl.when(wid == NW - 1)
    def _():
        pltpu.sync_copy(x_hbm.at[pl.ds(base, last_real), :],
                        xv.at[pl.ds(0, last_real), :])

    plsc.subcore_barrier()

    def sbody(g, carry):
        pltpu.sync_copy(xv.at[pl.ds(g * 128, 128), :], accs.at[idxv.at[g, :]],
                        add=True)
        pltpu.sync_copy(onesv, accc.at[idxv.at[g, :]], add=True)
        return carry

    lax.fori_loop(0, G, sbody, 0)
    plsc.subcore_barrier()

    pltpu.sync_copy(accs.at[pl.ds(sid * ROWS_PER_TILE, ROWS_PER_TILE), :],
                    out_s.at[cid, pl.ds(sid * ROWS_PER_TILE, ROWS_PER_TILE), :])
    pltpu.sync_copy(accc.at[pl.ds(sid * ROWS_PER_TILE, ROWS_PER_TILE), :],
                    out_c.at[cid, pl.ds(sid * ROWS_PER_TILE, ROWS_PER_TILE), :])


_seg_kernel = pl.kernel(
    _seg_body,
    out_type=(
        jax.ShapeDtypeStruct((NC, NUM_GRAPHS, HIDDEN), jnp.float32),
        jax.ShapeDtypeStruct((NC, NUM_GRAPHS, 1), jnp.float32),
    ),
    mesh=plsc.VectorSubcoreMesh(core_axis_name="c", subcore_axis_name="s",
                                num_cores=NC, num_subcores=NS),
    scratch_types=[
        pltpu.VMEM((CHUNK, HIDDEN), jnp.float32),
        pltpu.VMEM((G, 128), jnp.int32),
        pltpu.VMEM((128, 1), jnp.float32),
        pltpu.VMEM_SHARED((NUM_GRAPHS + 1, HIDDEN), jnp.float32),
        pltpu.VMEM_SHARED((NUM_GRAPHS + 1, 1), jnp.float32),
    ],
)


def _mlp_body(ps, pc, w1t, b1, w2t, b2, o):
    sums = ps[0] + ps[1]
    cnt = jnp.maximum(pc[0] + pc[1], 1.0)
    mean = sums / cnt
    h = jnp.maximum(
        jnp.dot(mean, w1t[...], preferred_element_type=jnp.float32) + b1[...], 0.0)
    o[...] = jnp.dot(h, w2t[...], preferred_element_type=jnp.float32) + b2[...]


def _mlp(ps, pc, w1t, b1, w2t, b2):
    return pl.pallas_call(
        _mlp_body,
        out_shape=jax.ShapeDtypeStruct((NUM_GRAPHS, 2), jnp.float32),
    )(ps, pc, w1t, b1, w2t, b2)


def kernel(x, edge_index, edge_attr, u, batch, W1, b1, W2, b2):
    idx = jnp.pad(batch.astype(jnp.int32), (0, N_PAD - N_NODES),
                  constant_values=NUM_GRAPHS).reshape(N_PAD // 128, 128)
    z14 = jnp.zeros((ROWS_PER_TILE, HIDDEN), jnp.float32)
    z1 = jnp.zeros((ROWS_PER_TILE, 1), jnp.float32)
    ones128 = jnp.ones((128, 1), jnp.float32)
    ps, pc = _seg_kernel(x, idx, z14, z1, ones128)
    return _mlp(ps, pc, W1.T, b1[None, :], W2.T, b2[None, :])


# speedup vs baseline: 5.8668x; 5.8668x over previous
import jax
import jax.numpy as jnp
from jax import lax
from jax.experimental import pallas as pl
from jax.experimental.pallas import tpu as pltpu
from jax.experimental.pallas import tpu_sc as plsc

NUM_GRAPHS = 1024
HIDDEN = 14
N_NODES = 100000
W = 16

NC = 2
NS = 16
NW = NC * NS
CHUNK = 3200
G = CHUNK // 128
N_PAD = NW * CHUNK
ROWS_PER_TILE = NUM_GRAPHS // NS


def _seg_body(x_hbm, idx_hbm, z_hbm, out_p, xv, idxv, acc):
    cid = lax.axis_index("c")
    sid = lax.axis_index("s")
    wid = cid * NS + sid
    base = wid * CHUNK

    pltpu.sync_copy(z_hbm, acc.at[pl.ds(sid * ROWS_PER_TILE, ROWS_PER_TILE), :])

    pltpu.sync_copy(idx_hbm.at[wid], idxv)

    last_real = N_NODES - (NW - 1) * CHUNK

    @pl.when(wid < NW - 1)
    def _():
        pltpu.sync_copy(x_hbm.at[pl.ds(base, CHUNK), :], xv.at[pl.ds(0, CHUNK), :])

    @pl.when(wid == NW - 1)
    def _():
        pltpu.sync_copy(x_hbm.at[pl.ds(base, last_real), :],
                        xv.at[pl.ds(0, last_real), :])

    plsc.subcore_barrier()

    def sbody(g, carry):
        pltpu.sync_copy(xv.at[pl.ds(g * 128, 128), :], acc.at[idxv.at[g, :]],
                        add=True)
        return carry

    lax.fori_loop(0, G, sbody, 0)
    plsc.subcore_barrier()

    pltpu.sync_copy(acc.at[pl.ds(sid * ROWS_PER_TILE, ROWS_PER_TILE), :],
                    out_p.at[cid, pl.ds(sid * ROWS_PER_TILE, ROWS_PER_TILE), :])


_seg_kernel = pl.kernel(
    _seg_body,
    out_type=jax.ShapeDtypeStruct((NC, NUM_GRAPHS, W), jnp.float32),
    mesh=plsc.VectorSubcoreMesh(core_axis_name="c", subcore_axis_name="s",
                                num_cores=NC, num_subcores=NS),
    scratch_types=[
        pltpu.VMEM((CHUNK, W), jnp.float32),
        pltpu.VMEM((G, 128), jnp.int32),
        pltpu.VMEM_SHARED((NUM_GRAPHS + 1, W), jnp.float32),
    ],
    compiler_params=pltpu.CompilerParams(use_tc_tiling_on_sc=False),
)


def _mlp_body(p, w1t, b1, w2t, b2, o):
    a = p[0] + p[1]
    cnt = jnp.maximum(a[:, HIDDEN:HIDDEN + 1], 1.0)
    mean = a / cnt
    h = jnp.maximum(
        jnp.dot(mean, w1t[...], preferred_element_type=jnp.float32) + b1[...], 0.0)
    o[...] = jnp.dot(h, w2t[...], preferred_element_type=jnp.float32) + b2[...]


def _mlp(p, w1t, b1, w2t, b2):
    return pl.pallas_call(
        _mlp_body,
        out_shape=jax.ShapeDtypeStruct((NUM_GRAPHS, 2), jnp.float32),
    )(p, w1t, b1, w2t, b2)


def kernel(x, edge_index, edge_attr, u, batch, W1, b1, W2, b2):
    xp = jnp.concatenate(
        [x, jnp.ones((N_NODES, 1), jnp.float32),
         jnp.zeros((N_NODES, 1), jnp.float32)], axis=1)
    idx = jnp.pad(batch.astype(jnp.int32), (0, N_PAD - N_NODES),
                  constant_values=NUM_GRAPHS).reshape(NW, G, 128)
    z = jnp.zeros((ROWS_PER_TILE, W), jnp.float32)
    p = _seg_kernel(xp, idx, z)
    w1t = jnp.concatenate([W1.T, jnp.zeros((W - HIDDEN, HIDDEN), jnp.float32)],
                          axis=0)
    return _mlp(p, w1t, b1[None, :], W2.T, b2[None, :])


# speedup vs baseline: 5.9732x; 1.0181x over previous
import jax
import jax.numpy as jnp
from jax import lax
from jax.experimental import pallas as pl
from jax.experimental.pallas import tpu as pltpu
from jax.experimental.pallas import tpu_sc as plsc

NUM_GRAPHS = 1024
HIDDEN = 14
N_NODES = 100000
W = 16

NC = 2
NS = 16
NW = NC * NS
CHUNK = 3200
G = CHUNK // 128
N_PAD = NW * CHUNK
ROWS_PER_TILE = NUM_GRAPHS // NS


def _seg_body(x_hbm, idx_hbm, z_hbm, out_p, xv, idxv, acc, sem):
    cid = lax.axis_index("c")
    sid = lax.axis_index("s")
    wid = cid * NS + sid
    base = wid * CHUNK
    last_real = N_NODES - (NW - 1) * CHUNK

    pend = [
        pltpu.async_copy(
            z_hbm, acc.at[pl.ds(sid * ROWS_PER_TILE, ROWS_PER_TILE), :], sem),
        pltpu.async_copy(idx_hbm.at[wid], idxv, sem),
    ]

    @pl.when(wid < NW - 1)
    def _():
        pltpu.async_copy(x_hbm.at[pl.ds(base, CHUNK), :],
                         xv.at[pl.ds(0, CHUNK), :], sem).wait()

    @pl.when(wid == NW - 1)
    def _():
        pltpu.async_copy(x_hbm.at[pl.ds(base, last_real), :],
                         xv.at[pl.ds(0, last_real), :], sem).wait()

    for h in pend:
        h.wait()

    plsc.subcore_barrier()

    scat = [
        pltpu.async_copy(xv.at[pl.ds(g * 128, 128), :], acc.at[idxv.at[g, :]],
                         sem, add=True)
        for g in range(G)
    ]
    for h in scat:
        h.wait()

    plsc.subcore_barrier()

    pltpu.sync_copy(acc.at[pl.ds(sid * ROWS_PER_TILE, ROWS_PER_TILE), :],
                    out_p.at[cid, pl.ds(sid * ROWS_PER_TILE, ROWS_PER_TILE), :])


_seg_kernel = pl.kernel(
    _seg_body,
    out_type=jax.ShapeDtypeStruct((NC, NUM_GRAPHS, W), jnp.float32),
    mesh=plsc.VectorSubcoreMesh(core_axis_name="c", subcore_axis_name="s",
                                num_cores=NC, num_subcores=NS),
    scratch_types=[
        pltpu.VMEM((CHUNK, W), jnp.float32),
        pltpu.VMEM((G, 128), jnp.int32),
        pltpu.VMEM_SHARED((NUM_GRAPHS + 1, W), jnp.float32),
        pltpu.SemaphoreType.DMA,
    ],
    compiler_params=pltpu.CompilerParams(use_tc_tiling_on_sc=False),
)


def _mlp_body(p, w1t, b1, w2t, b2, o):
    a = p[0] + p[1]
    cnt = jnp.maximum(a[:, HIDDEN:HIDDEN + 1], 1.0)
    mean = a[:, :HIDDEN] / cnt
    h = jnp.maximum(
        jnp.dot(mean, w1t[...], preferred_element_type=jnp.float32) + b1[...], 0.0)
    o[...] = jnp.dot(h, w2t[...], preferred_element_type=jnp.float32) + b2[...]


def _mlp(p, w1t, b1, w2t, b2):
    return pl.pallas_call(
        _mlp_body,
        out_shape=jax.ShapeDtypeStruct((NUM_GRAPHS, 2), jnp.float32),
    )(p, w1t, b1, w2t, b2)


def kernel(x, edge_index, edge_attr, u, batch, W1, b1, W2, b2):
    xp = jnp.concatenate(
        [x, jnp.ones((N_NODES, 1), jnp.float32),
         jnp.zeros((N_NODES, 1), jnp.float32)], axis=1)
    idx = jnp.pad(batch.astype(jnp.int32), (0, N_PAD - N_NODES),
                  constant_values=NUM_GRAPHS).reshape(NW, G, 128)
    z = jnp.zeros((ROWS_PER_TILE, W), jnp.float32)
    p = _seg_kernel(xp, idx, z)
    return _mlp(p, W1.T, b1[None, :], W2.T, b2[None, :])
